# baseline (speedup 1.0000x reference)
import functools

import numpy as np
import jax
import jax.numpy as jnp
from jax import lax
from jax.experimental import pallas as pl
from jax.experimental.pallas import tpu as pltpu
from jax.experimental.pallas import tpu_sc as plsc

N_BATCH = 16384
N_PARTICLES = 256
N_DIM = 3
N_CG = N_PARTICLES // 4
N_NCG = N_PARTICLES - N_CG

NW = 32
BLOCKS_PER_W = 4
NSLAB = BLOCKS_PER_W * N_DIM

IN_PLANE = N_BATCH * N_PARTICLES
SLAB = 16 * 2 * 8 * 128
HSLAB_ROWS = 128
PITCH = 69


def _sc_split(x_flat):
    mesh = plsc.VectorSubcoreMesh(core_axis_name="c", subcore_axis_name="s")

    @functools.partial(
        pl.kernel,
        mesh=mesh,
        compiler_params=pltpu.CompilerParams(needs_layout_passes=False),
        out_type=(
            jax.ShapeDtypeStruct((N_DIM * 8, 128, 8, 128), jnp.float32),
            jax.ShapeDtypeStruct((N_DIM * 24, 128, 8, 128), jnp.float32),
        ),
        scratch_types=[
            pltpu.VMEM((HSLAB_ROWS, 128), jnp.float32),
            pltpu.VMEM((HSLAB_ROWS, 128), jnp.float32),
            pltpu.VMEM((HSLAB_ROWS, 128), jnp.float32),
            pltpu.VMEM((N_PARTICLES * PITCH,), jnp.float32),
            pltpu.VMEM((8, 8, 128), jnp.float32),
            pltpu.VMEM((24, 8, 128), jnp.float32),
            pltpu.SemaphoreType.DMA,
            pltpu.SemaphoreType.DMA,
            pltpu.SemaphoreType.DMA,
            pltpu.SemaphoreType.DMA,
            pltpu.SemaphoreType.DMA,
        ],
    )
    def k(x_hbm, cg_hbm, ncg_hbm, vin0, vin1, vin2, vtmp, vcg, vncg,
          sin0, sin1, sin2, scg, sncg):
        wid = lax.axis_index("s") * 2 + lax.axis_index("c")
        vins = [vin0, vin1, vin2]
        sins = [sin0, sin1, sin2]

        ii = lax.iota(jnp.int32, 16)
        a, m = ii // 4, ii % 4
        idx69 = []
        for qv in range(16):
            q4 = 4 * qv + a
            row = jnp.where(m == 0, q4, 64 + 3 * q4 + m - 1)
            idx69.append(row * PITCH)

        def coords(sl):
            c = sl // 3
            p = sl - c * 3
            tbo = wid * BLOCKS_PER_W + c
            return p, tbo

        def start_in(sl, half, buf):
            p, tbo = coords(sl)
            row0 = pl.multiple_of(
                p * (IN_PLANE // 128) + tbo * (SLAB // 128) + half * 128, 128)
            return pltpu.async_copy(
                x_hbm.at[pl.ds(row0, HSLAB_ROWS)], vins[buf], sins[buf])

        def wait_in(buf):
            pltpu.make_async_copy(
                x_hbm.at[pl.ds(0, HSLAB_ROWS)], vins[buf], sins[buf]).wait()

        def wait_cg():
            pltpu.make_async_copy(
                vcg, cg_hbm.at[pl.ds(0, 8), 0], scg).wait()

        def wait_ncg():
            pltpu.make_async_copy(
                vncg, ncg_hbm.at[pl.ds(0, 24), 0], sncg).wait()

        def do_half(buf, half):
            vin = vins[buf]

            @plsc.parallel_loop(0, 64, unroll=8)
            def s1_body(bl):
                tb16 = (bl // 8) * 16 + bl % 8
                for qv in range(16):
                    rowb = tb16 + (qv // 8) * 8
                    val = vin[rowb, pl.ds((qv % 8) * 16, 16)]
                    plsc.store_scatter(vtmp, [idx69[qv] + bl], val)

            @plsc.parallel_loop(0, 64, unroll=8)
            def s2cg_body(r):
                for c in range(4):
                    vcg[r // 8, r % 8, pl.ds(half * 64 + 16 * c, 16)] = (
                        vtmp[pl.ds(r * PITCH + 16 * c, 16)])

            @plsc.parallel_loop(0, 192, unroll=8)
            def s2ncg_body(r):
                for c in range(4):
                    vncg[r // 8, r % 8, pl.ds(half * 64 + 16 * c, 16)] = (
                        vtmp[pl.ds((64 + r) * PITCH + 16 * c, 16)])

        start_in(0, 0, 0)
        start_in(0, 1, 1)

        def six_body(it, _):
            sl0 = it * 3
            for j in range(6):
                i_half = sl0 * 2 + j
                sl = sl0 + j // 2
                half = j % 2
                wait_in(j % 3)
                sl2 = jnp.minimum(sl0 + (j + 2) // 2, NSLAB - 1)
                start_in(sl2, (j + 2) % 2, (j + 2) % 3)
                if half == 0:
                    @pl.when(sl > 0)
                    def _drain_outs():
                        wait_cg()
                        wait_ncg()
                do_half(j % 3, half)
                if half == 1:
                    p, tbo = coords(sl)
                    pltpu.async_copy(vcg, cg_hbm.at[pl.ds(p * 8, 8), tbo],
                                     scg)
                    pltpu.async_copy(vncg, ncg_hbm.at[pl.ds(p * 24, 24), tbo],
                                     sncg)
            return _

        lax.fori_loop(0, NSLAB // 3, six_body, None)
        wait_cg()
        wait_ncg()
        wait_in(0)
        wait_in(1)

    return k(x_flat)


@jax.jit
def kernel(input_coords):
    x5 = (input_coords.transpose(2, 0, 1)
          .reshape(N_DIM, 2048, 8, 2, 128)
          .transpose(0, 1, 3, 2, 4))
    x_flat = x5.reshape(N_DIM * IN_PLANE // 128, 128)
    cg_t, ncg_t = _sc_split(x_flat)
    cg = (cg_t.reshape(N_DIM, 8, 128, 8, 128)
          .transpose(0, 1, 3, 2, 4)
          .reshape(N_DIM, N_CG, N_BATCH)
          .transpose(2, 1, 0))
    ncg = (ncg_t.reshape(N_DIM, 24, 128, 8, 128)
           .transpose(0, 1, 3, 2, 4)
           .reshape(N_DIM, N_NCG, N_BATCH)
           .transpose(2, 1, 0))
    return (cg, ncg)


# speedup vs baseline: 1.1022x; 1.1022x over previous
import functools

import numpy as np
import jax
import jax.numpy as jnp
from jax import lax
from jax.experimental import pallas as pl
from jax.experimental.pallas import tpu as pltpu
from jax.experimental.pallas import tpu_sc as plsc

N_BATCH = 16384
N_PARTICLES = 256
N_DIM = 3
N_CG = N_PARTICLES // 4
N_NCG = N_PARTICLES - N_CG

NW = 32
BLOCKS_PER_W = 4
NSLAB = BLOCKS_PER_W * N_DIM

IN_PLANE = N_BATCH * N_PARTICLES
SLAB = 16 * 2 * 8 * 128
HSLAB_ROWS = 128
PITCH = 69


def _sc_split(x_flat):
    mesh = plsc.VectorSubcoreMesh(core_axis_name="c", subcore_axis_name="s")

    @functools.partial(
        pl.kernel,
        mesh=mesh,
        compiler_params=pltpu.CompilerParams(needs_layout_passes=False),
        out_type=(
            jax.ShapeDtypeStruct((N_DIM * 8, 128, 8, 128), jnp.float32),
            jax.ShapeDtypeStruct((N_DIM * 24, 128, 8, 128), jnp.float32),
        ),
        scratch_types=[
            pltpu.VMEM((HSLAB_ROWS, 128), jnp.float32),
            pltpu.VMEM((HSLAB_ROWS, 128), jnp.float32),
            pltpu.VMEM((N_PARTICLES * PITCH,), jnp.float32),
            pltpu.VMEM((8, 8, 128), jnp.float32),
            pltpu.VMEM((24, 8, 128), jnp.float32),
            pltpu.SemaphoreType.DMA,
            pltpu.SemaphoreType.DMA,
            pltpu.SemaphoreType.DMA,
            pltpu.SemaphoreType.DMA,
        ],
    )
    def k(x_hbm, cg_hbm, ncg_hbm, vin0, vin1, vtmp, vcg, vncg,
          sin0, sin1, scg, sncg):
        wid = lax.axis_index("s") * 2 + lax.axis_index("c")
        vins = [vin0, vin1]
        sins = [sin0, sin1]

        ii = lax.iota(jnp.int32, 16)
        a, m = ii // 4, ii % 4
        idx69 = []
        for qv in range(16):
            q4 = 4 * qv + a
            row = jnp.where(m == 0, q4, 64 + 3 * q4 + m - 1)
            idx69.append(row * PITCH)

        def coords(sl):
            c = sl // 3
            p = sl - c * 3
            tbo = wid * BLOCKS_PER_W + c
            return p, tbo

        def start_in(sl, half):
            p, tbo = coords(sl)
            row0 = pl.multiple_of(
                p * (IN_PLANE // 128) + tbo * (SLAB // 128) + half * 128, 128)
            return pltpu.async_copy(
                x_hbm.at[pl.ds(row0, HSLAB_ROWS)], vins[half], sins[half])

        def wait_in(half):
            pltpu.make_async_copy(
                x_hbm.at[pl.ds(0, HSLAB_ROWS)], vins[half], sins[half]).wait()

        def wait_cg():
            pltpu.make_async_copy(
                vcg, cg_hbm.at[pl.ds(0, 8), 0], scg).wait()

        def wait_ncg():
            pltpu.make_async_copy(
                vncg, ncg_hbm.at[pl.ds(0, 24), 0], sncg).wait()

        def do_half(sl, half):
            vin = vins[half]

            @plsc.parallel_loop(0, 64, unroll=8)
            def s1_body(bl):
                tb16 = (bl // 8) * 16 + bl % 8
                for qv in range(16):
                    rowb = tb16 + (qv // 8) * 8
                    val = vin[rowb, pl.ds((qv % 8) * 16, 16)]
                    plsc.store_scatter(vtmp, [idx69[qv] + bl], val)

            @plsc.parallel_loop(0, 64, unroll=8)
            def s2cg_body(r):
                for c in range(4):
                    vcg[r // 8, r % 8, pl.ds(half * 64 + 16 * c, 16)] = (
                        vtmp[pl.ds(r * PITCH + 16 * c, 16)])

            @plsc.parallel_loop(0, 192, unroll=8)
            def s2ncg_body(r):
                for c in range(4):
                    vncg[r // 8, r % 8, pl.ds(half * 64 + 16 * c, 16)] = (
                        vtmp[pl.ds((64 + r) * PITCH + 16 * c, 16)])

        start_in(0, 0)

        def slab_body(sl, _):
            wait_in(0)
            start_in(sl, 1)

            @pl.when(sl > 0)
            def _drain_outs():
                wait_cg()
                wait_ncg()

            do_half(sl, 0)

            wait_in(1)
            nxt = jnp.minimum(sl + 1, NSLAB - 1)
            start_in(nxt, 0)
            do_half(sl, 1)

            p, tbo = coords(sl)
            pltpu.async_copy(vcg, cg_hbm.at[pl.ds(p * 8, 8), tbo], scg)
            pltpu.async_copy(vncg, ncg_hbm.at[pl.ds(p * 24, 24), tbo], sncg)
            return _

        lax.fori_loop(0, NSLAB, slab_body, None)
        wait_cg()
        wait_ncg()
        wait_in(0)

    return k(x_flat)


@jax.jit
def kernel(input_coords):
    x5 = (input_coords.transpose(2, 0, 1)
          .reshape(N_DIM, 2048, 8, 2, 128)
          .transpose(0, 1, 3, 2, 4))
    x_flat = x5.reshape(N_DIM * IN_PLANE // 128, 128)
    cg_t, ncg_t = _sc_split(x_flat)
    cg = (cg_t.reshape(N_DIM, 8, 128, 8, 128)
          .transpose(0, 1, 3, 2, 4)
          .reshape(N_DIM, N_CG, N_BATCH)
          .transpose(2, 1, 0))
    ncg = (ncg_t.reshape(N_DIM, 24, 128, 8, 128)
           .transpose(0, 1, 3, 2, 4)
           .reshape(N_DIM, N_NCG, N_BATCH)
           .transpose(2, 1, 0))
    return (cg, ncg)
